# Initial kernel scaffold; baseline (speedup 1.0000x reference)
import jax
import jax.numpy as jnp
from jax.experimental import pallas as pl


def kernel(x, edge_index, Wl, bl, Wr, W2, b2, gamma, beta, Wo, bo):
    raise NotImplementedError("write your pallas kernel here")


# speedup vs baseline: 4.7054x; 4.7054x over previous
import jax
import jax.numpy as jnp
from jax import lax
from jax.experimental import pallas as pl
from jax.experimental.pallas import tpu as pltpu
from jax.experimental.pallas import tpu_sc as plsc

N = 10000
NPAD = 10240
E = 320000
D = 128

NC = 2
NS = 16
NW = NC * NS
EPW = E // NW
CK = 80
NCHUNK = EPW // CK
RPT = NPAD // NS


def _sc_aggregate_body(x_hbm, src_hbm, dst_hbm, zrow_hbm, onerow_hbm,
                       agg_out, cnt_out,
                       agg_sh, sidx_v, didx_v, rows_v, sem):
  c = lax.axis_index("c")
  s = lax.axis_index("s")
  wid = s * NC + c
  r0 = s * RPT

  def zero_shared():
    pltpu.sync_copy(zrow_hbm, rows_v)
    for j in range(RPT // CK):
      pltpu.sync_copy(rows_v, agg_sh.at[pl.ds(r0 + j * CK, CK)])

  def write_out(dst_hbm_3d):
    for j in range(RPT // CK):
      o = r0 + j * CK
      pltpu.sync_copy(agg_sh.at[pl.ds(o, CK)], rows_v)
      pltpu.sync_copy(rows_v, dst_hbm_3d.at[c, pl.ds(o, CK)])

  zero_shared()
  plsc.subcore_barrier()

  def step(i, _):
    base = wid * EPW + i * CK
    pltpu.sync_copy(src_hbm.at[pl.ds(base, CK)], sidx_v)
    pltpu.sync_copy(dst_hbm.at[pl.ds(base, CK)], didx_v)
    pltpu.async_copy(x_hbm.at[sidx_v], rows_v, sem).wait()
    pltpu.sync_copy(rows_v, agg_sh.at[didx_v], add=True)
    return 0
  lax.fori_loop(0, NCHUNK, step, 0)
  plsc.subcore_barrier()
  write_out(agg_out)
  plsc.subcore_barrier()

  zero_shared()
  plsc.subcore_barrier()
  pltpu.sync_copy(onerow_hbm, rows_v)

  def step_cnt(i, _):
    base = wid * EPW + i * CK
    pltpu.sync_copy(dst_hbm.at[pl.ds(base, CK)], didx_v)
    pltpu.sync_copy(rows_v, agg_sh.at[didx_v], add=True)
    return 0
  lax.fori_loop(0, NCHUNK, step_cnt, 0)
  plsc.subcore_barrier()
  write_out(cnt_out)


@jax.jit
def _sc_aggregate(x, src, dst, zrow, onerow):
  mesh = plsc.VectorSubcoreMesh(core_axis_name="c", subcore_axis_name="s")
  return pl.kernel(
      _sc_aggregate_body,
      out_type=(
          jax.ShapeDtypeStruct((NC, NPAD, D), jnp.float32),
          jax.ShapeDtypeStruct((NC, NPAD, D), jnp.float32),
      ),
      mesh=mesh,
      scratch_types=[
          pltpu.VMEM_SHARED((NPAD, D), jnp.float32),
          pltpu.VMEM((CK,), jnp.int32),
          pltpu.VMEM((CK,), jnp.int32),
          pltpu.VMEM((CK, D), jnp.float32),
          pltpu.SemaphoreType.DMA,
      ],
  )(x, src, dst, zrow, onerow)


def _dense_body(x_ref, agg_ref, cnt_ref, Wl_ref, Wr_ref, W2_ref, bl_ref,
                b2_ref, gamma_ref, beta_ref, Wo_ref, bo_ref, out_ref):
  agg = agg_ref[0, :N] + agg_ref[1, :N]
  cnt = cnt_ref[0] + cnt_ref[1]
  agg = agg / jnp.maximum(cnt, 1.0)
  Wc = Wr_ref[...] + W2_ref[...]
  bc = (bl_ref[...] + b2_ref[...])[None, :]
  h = (lax.dot_general(agg, Wl_ref[...], (((1,), (1,)), ((), ())),
                       preferred_element_type=jnp.float32)
       + lax.dot_general(x_ref[...], Wc, (((1,), (1,)), ((), ())),
                         preferred_element_type=jnp.float32)
       + bc)
  mean = jnp.mean(h, axis=0, keepdims=True)
  hc = h - mean
  var = jnp.mean(hc * hc, axis=0, keepdims=True)
  hn = hc * lax.rsqrt(var + 1e-5) * gamma_ref[...][None, :] + beta_ref[...][None, :]
  hn = jnp.where(hn > 0, hn, jnp.exp(hn) - 1.0)
  out_ref[...] = (jnp.sum(hn * Wo_ref[...], axis=1, keepdims=True)
                  + bo_ref[0])


@jax.jit
def _dense(x, agg, cnt, Wl, Wr, W2, bl, b2, gamma, beta, Wo, bo):
  return pl.pallas_call(
      _dense_body,
      out_shape=jax.ShapeDtypeStruct((N, 1), jnp.float32),
  )(x, agg, cnt, Wl, Wr, W2, bl, b2, gamma, beta, Wo, bo)


def kernel(x, edge_index, Wl, bl, Wr, W2, b2, gamma, beta, Wo, bo):
  src = edge_index[0]
  dst = edge_index[1]
  zrow = jnp.zeros((CK, D), jnp.float32)
  onerow = jnp.ones((CK, D), jnp.float32)
  agg, cnt = _sc_aggregate(x, src, dst, zrow, onerow)
  cnt_col = cnt[:, :N, :1]
  return _dense(x, agg, cnt_col, Wl, Wr, W2, bl, b2, gamma, beta, Wo, bo)


# speedup vs baseline: 9.0042x; 1.9136x over previous
import jax
import jax.numpy as jnp
from jax import lax
from jax.experimental import pallas as pl
from jax.experimental.pallas import tpu as pltpu
from jax.experimental.pallas import tpu_sc as plsc

N = 10000
NPAD = 10240
E = 320000
D = 128

NC = 2
NS = 16
NW = NC * NS
EPW = E // NW
CK = 40
G = 5
GCH = 50
NCHUNK = G * GCH
RPT = NPAD // NS
ZB = 80
R = 5


def _sc_aggregate_body(x_hbm, src_hbm, dst_hbm, zrow_hbm, onerow_hbm,
                       agg_out, cnt_out,
                       agg_sh, sidx_v, didx_v, rows_v, zbuf_v,
                       gs0, gs1, gs2, gs3, gs4, ss0, ss1, ss2, ss3, ss4):
  c = lax.axis_index("c")
  s = lax.axis_index("s")
  wid = s * NC + c
  r0 = s * RPT
  gsem = [gs0, gs1, gs2, gs3, gs4]
  ssem = [ss0, ss1, ss2, ss3, ss4]

  def gather(u, j):
    return pltpu.make_async_copy(
        x_hbm.at[sidx_v.at[pl.ds(u * CK, CK)]], rows_v.at[j], gsem[j])

  def scat(src_ref, u, j):
    return pltpu.make_async_copy(src_ref, agg_sh.at[didx_v.at[u]], ssem[j])

  def zero_shared():
    pltpu.sync_copy(zrow_hbm, zbuf_v)
    for j in range(RPT // ZB):
      pltpu.sync_copy(zbuf_v, agg_sh.at[pl.ds(r0 + j * ZB, ZB)])

  def write_out(dst_hbm_3d):
    for j in range(RPT // ZB):
      o = r0 + j * ZB
      pltpu.sync_copy(agg_sh.at[pl.ds(o, ZB)], zbuf_v)
      pltpu.sync_copy(zbuf_v, dst_hbm_3d.at[c, pl.ds(o, ZB)])

  zero_shared()
  plsc.subcore_barrier()

  def run_group(g, count_phase):
    pltpu.sync_copy(dst_hbm.at[wid, g], didx_v)
    if not count_phase:
      pltpu.sync_copy(src_hbm.at[pl.ds((wid * G + g) * GCH * CK, GCH * CK)],
                      sidx_v)
      for j in range(R):
        gather(j, j).start()

      def step(k, _):
        for j in range(R):
          u = k * R + j
          jm2 = (j - 2) % R

          @pl.when(k > 0)
          def _recycle():
            scat(rows_v.at[j], u - R, j).wait()
            gather(u, j).start()

          def _drain():
            gather(u - 2, jm2).wait()
            scat(rows_v.at[jm2], u - 2, jm2).start(add=True)
          if j < 2:
            pl.when(k > 0)(_drain)
          else:
            _drain()
        return 0
      lax.fori_loop(0, GCH // R, step, 0)
      for u in (GCH - 2, GCH - 1):
        j = u % R
        gather(u, j).wait()
        scat(rows_v.at[j], u, j).start(add=True)
    else:
      def step_cnt(k, _):
        for j in range(R):
          u = k * R + j

          @pl.when(k > 0)
          def _wait_prev():
            scat(rows_v.at[0], u - R, j).wait()
          scat(rows_v.at[0], u, j).start(add=True)
        return 0
      lax.fori_loop(0, GCH // R, step_cnt, 0)
    for u in range(GCH - R, GCH):
      scat(rows_v.at[0], u, u % R).wait()

  for g in range(G):
    run_group(g, count_phase=False)
  plsc.subcore_barrier()
  write_out(agg_out)
  plsc.subcore_barrier()

  zero_shared()
  plsc.subcore_barrier()
  pltpu.sync_copy(onerow_hbm, rows_v.at[0])
  for g in range(G):
    run_group(g, count_phase=True)
  plsc.subcore_barrier()
  write_out(cnt_out)


@jax.jit
def _sc_aggregate(x, src, dst, zrow, onerow):
  mesh = plsc.VectorSubcoreMesh(core_axis_name="c", subcore_axis_name="s")
  return pl.kernel(
      _sc_aggregate_body,
      out_type=(
          jax.ShapeDtypeStruct((NC, NPAD, D), jnp.float32),
          jax.ShapeDtypeStruct((NC, NPAD, D), jnp.float32),
      ),
      mesh=mesh,
      scratch_types=[
          pltpu.VMEM_SHARED((NPAD, D), jnp.float32),
          pltpu.VMEM((GCH * CK,), jnp.int32),
          pltpu.VMEM((GCH, CK), jnp.int32),
          pltpu.VMEM((R, CK, D), jnp.float32),
          pltpu.VMEM((ZB, D), jnp.float32),
          pltpu.SemaphoreType.DMA,
          pltpu.SemaphoreType.DMA,
          pltpu.SemaphoreType.DMA,
          pltpu.SemaphoreType.DMA,
          pltpu.SemaphoreType.DMA,
          pltpu.SemaphoreType.DMA,
          pltpu.SemaphoreType.DMA,
          pltpu.SemaphoreType.DMA,
          pltpu.SemaphoreType.DMA,
          pltpu.SemaphoreType.DMA,
      ],
  )(x, src, dst, zrow, onerow)


def _dense_body(x_ref, agg_ref, cnt_ref, Wl_ref, Wr_ref, W2_ref, bl_ref,
                b2_ref, gamma_ref, beta_ref, Wo_ref, bo_ref, out_ref):
  agg = agg_ref[0, :N] + agg_ref[1, :N]
  cnt = cnt_ref[0] + cnt_ref[1]
  agg = agg / jnp.maximum(cnt, 1.0)
  Wc = Wr_ref[...] + W2_ref[...]
  bc = (bl_ref[...] + b2_ref[...])[None, :]
  h = (lax.dot_general(agg, Wl_ref[...], (((1,), (1,)), ((), ())),
                       preferred_element_type=jnp.float32)
       + lax.dot_general(x_ref[...], Wc, (((1,), (1,)), ((), ())),
                         preferred_element_type=jnp.float32)
       + bc)
  mean = jnp.mean(h, axis=0, keepdims=True)
  hc = h - mean
  var = jnp.mean(hc * hc, axis=0, keepdims=True)
  hn = hc * lax.rsqrt(var + 1e-5) * gamma_ref[...][None, :] + beta_ref[...][None, :]
  hn = jnp.where(hn > 0, hn, jnp.exp(hn) - 1.0)
  out_ref[...] = (jnp.sum(hn * Wo_ref[...], axis=1, keepdims=True)
                  + bo_ref[0])


@jax.jit
def _dense(x, agg, cnt, Wl, Wr, W2, bl, b2, gamma, beta, Wo, bo):
  return pl.pallas_call(
      _dense_body,
      out_shape=jax.ShapeDtypeStruct((N, 1), jnp.float32),
  )(x, agg, cnt, Wl, Wr, W2, bl, b2, gamma, beta, Wo, bo)


def kernel(x, edge_index, Wl, bl, Wr, W2, b2, gamma, beta, Wo, bo):
  src = edge_index[0]
  dst = edge_index[1].reshape(NW, G, GCH, CK)
  zrow = jnp.zeros((ZB, D), jnp.float32)
  onerow = jnp.ones((CK, D), jnp.float32)
  agg, cnt = _sc_aggregate(x, src, dst, zrow, onerow)
  cnt_col = cnt[:, :N, :1]
  return _dense(x, agg, cnt_col, Wl, Wr, W2, bl, b2, gamma, beta, Wo, bo)


# speedup vs baseline: 9.2618x; 1.0286x over previous
import jax
import jax.numpy as jnp
from jax import lax
from jax.experimental import pallas as pl
from jax.experimental.pallas import tpu as pltpu
from jax.experimental.pallas import tpu_sc as plsc

N = 10000
NPAD = 10240
E = 320000
D = 128

NC = 2
NS = 16
NW = NC * NS
EPW = E // NW
CK = 40
G = 5
GCH = 50
NCHUNK = G * GCH
RPT = NPAD // NS
ZB = 80
R = 5


def _sc_aggregate_body(x_hbm, src_hbm, dst_hbm, zrow_hbm, onerow_hbm,
                       agg_out, cnt_out,
                       agg_sh, sidx_v, didx_v, rows_v, zbuf_v,
                       gs0, gs1, gs2, gs3, gs4, ss0, ss1, ss2, ss3, ss4):
  c = lax.axis_index("c")
  s = lax.axis_index("s")
  wid = s * NC + c
  r0 = s * RPT
  gsem = [gs0, gs1, gs2, gs3, gs4]
  ssem = [ss0, ss1, ss2, ss3, ss4]

  def gather(u, j):
    return pltpu.make_async_copy(
        x_hbm.at[sidx_v.at[pl.ds(u * CK, CK)]], rows_v.at[j], gsem[j])

  def scat(src_ref, u, j):
    return pltpu.make_async_copy(src_ref, agg_sh.at[didx_v.at[u]], ssem[j])

  def zero_shared():
    pltpu.sync_copy(zrow_hbm, zbuf_v)
    for j in range(RPT // ZB):
      pltpu.sync_copy(zbuf_v, agg_sh.at[pl.ds(r0 + j * ZB, ZB)])

  def write_out(dst_hbm_3d):
    for j in range(RPT // ZB):
      o = r0 + j * ZB
      pltpu.sync_copy(agg_sh.at[pl.ds(o, ZB)], zbuf_v)
      pltpu.sync_copy(zbuf_v, dst_hbm_3d.at[c, pl.ds(o, ZB)])

  zero_shared()
  plsc.subcore_barrier()

  def run_group(g, count_phase):
    pltpu.sync_copy(dst_hbm.at[wid, g], didx_v)
    if not count_phase:
      pltpu.sync_copy(src_hbm.at[pl.ds((wid * G + g) * GCH * CK, GCH * CK)],
                      sidx_v)
      for j in range(R):
        gather(j, j).start()

      def step(k, _):
        for j in range(R):
          u = k * R + j
          jm2 = (j - 2) % R

          @pl.when(k > 0)
          def _recycle():
            scat(rows_v.at[j], u - R, j).wait()
            gather(u, j).start()

          def _drain():
            gather(u - 2, jm2).wait()
            scat(rows_v.at[jm2], u - 2, jm2).start(add=True)
          if j < 2:
            pl.when(k > 0)(_drain)
          else:
            _drain()
        return 0
      lax.fori_loop(0, GCH // R, step, 0)
      for u in (GCH - 2, GCH - 1):
        j = u % R
        gather(u, j).wait()
        scat(rows_v.at[j], u, j).start(add=True)
    else:
      def step_cnt(k, _):
        for j in range(R):
          u = k * R + j

          @pl.when(k > 0)
          def _wait_prev():
            scat(rows_v.at[0], u - R, j).wait()
          scat(rows_v.at[0], u, j).start(add=True)
        return 0
      lax.fori_loop(0, GCH // R, step_cnt, 0)
    for u in range(GCH - R, GCH):
      scat(rows_v.at[0], u, u % R).wait()

  for g in range(G):
    run_group(g, count_phase=False)
  plsc.subcore_barrier()
  write_out(agg_out)
  plsc.subcore_barrier()

  zero_shared()
  plsc.subcore_barrier()
  pltpu.sync_copy(onerow_hbm, rows_v.at[0])
  for g in range(G):
    run_group(g, count_phase=True)
  plsc.subcore_barrier()
  write_out(cnt_out)


@jax.jit
def _sc_aggregate(x, src, dst, zrow, onerow):
  mesh = plsc.VectorSubcoreMesh(core_axis_name="c", subcore_axis_name="s")
  return pl.kernel(
      _sc_aggregate_body,
      out_type=(
          jax.ShapeDtypeStruct((NC, NPAD, D), jnp.float32),
          jax.ShapeDtypeStruct((NC, NPAD, D), jnp.float32),
      ),
      mesh=mesh,
      scratch_types=[
          pltpu.VMEM_SHARED((NPAD, D), jnp.float32),
          pltpu.VMEM((GCH * CK,), jnp.int32),
          pltpu.VMEM((GCH, CK), jnp.int32),
          pltpu.VMEM((R, CK, D), jnp.float32),
          pltpu.VMEM((ZB, D), jnp.float32),
          pltpu.SemaphoreType.DMA,
          pltpu.SemaphoreType.DMA,
          pltpu.SemaphoreType.DMA,
          pltpu.SemaphoreType.DMA,
          pltpu.SemaphoreType.DMA,
          pltpu.SemaphoreType.DMA,
          pltpu.SemaphoreType.DMA,
          pltpu.SemaphoreType.DMA,
          pltpu.SemaphoreType.DMA,
          pltpu.SemaphoreType.DMA,
      ],
  )(x, src, dst, zrow, onerow)


def _dense_body(x_ref, agg_ref, cnt_ref, Wl_ref, Wr_ref, W2_ref, bl_ref,
                b2_ref, gamma_ref, beta_ref, Wo_ref, bo_ref, out_ref):
  agg = agg_ref[0, :N] + agg_ref[1, :N]
  cnt = cnt_ref[0, :N, 0:1] + cnt_ref[1, :N, 0:1]
  agg = agg / jnp.maximum(cnt, 1.0)
  Wc = Wr_ref[...] + W2_ref[...]
  bc = (bl_ref[...] + b2_ref[...])[None, :]
  h = (lax.dot_general(agg, Wl_ref[...], (((1,), (1,)), ((), ())),
                       preferred_element_type=jnp.float32)
       + lax.dot_general(x_ref[...], Wc, (((1,), (1,)), ((), ())),
                         preferred_element_type=jnp.float32)
       + bc)
  mean = jnp.mean(h, axis=0, keepdims=True)
  hc = h - mean
  var = jnp.mean(hc * hc, axis=0, keepdims=True)
  hn = hc * lax.rsqrt(var + 1e-5) * gamma_ref[...][None, :] + beta_ref[...][None, :]
  hn = jnp.where(hn > 0, hn, jnp.exp(hn) - 1.0)
  out_ref[...] = (jnp.sum(hn * Wo_ref[...], axis=1, keepdims=True)
                  + bo_ref[0])


@jax.jit
def _dense(x, agg, cnt, Wl, Wr, W2, bl, b2, gamma, beta, Wo, bo):
  return pl.pallas_call(
      _dense_body,
      out_shape=jax.ShapeDtypeStruct((N, 1), jnp.float32),
  )(x, agg, cnt, Wl, Wr, W2, bl, b2, gamma, beta, Wo, bo)


def kernel(x, edge_index, Wl, bl, Wr, W2, b2, gamma, beta, Wo, bo):
  src = edge_index[0]
  dst = edge_index[1].reshape(NW, G, GCH, CK)
  zrow = jnp.zeros((ZB, D), jnp.float32)
  onerow = jnp.ones((CK, D), jnp.float32)
  agg, cnt = _sc_aggregate(x, src, dst, zrow, onerow)
  return _dense(x, agg, cnt, Wl, Wr, W2, bl, b2, gamma, beta, Wo, bo)
